# baseline (speedup 1.0000x reference)
import functools

import jax
import jax.numpy as jnp
from jax.experimental import pallas as pl


def _film(x, cond, W0, b0, W1, b1, W2, b2):
    h = jax.nn.relu(cond @ W0 + b0)
    h = jax.nn.relu(h @ W1 + b1)
    gb = h @ W2 + b2
    g = gb[:, :x.shape[1]]
    b = gb[:, x.shape[1]:]
    return g * x + b


def _bn(x, g, b):
    m = jnp.mean(x, axis=0)
    v = jnp.var(x, axis=0)
    return (x - m) / jnp.sqrt(v + 1e-5) * g + b


def _interaction_block(ri0, ri1, rj0, rj1, nsigma=5):
    sig = jnp.logspace(-1.0, 1.0, nsigma)
    vi = 4.0 / 3.0 * jnp.pi * (ri1 ** 3 - ri0 ** 3)
    vj = 4.0 / 3.0 * jnp.pi * (rj1 ** 3 - rj0 ** 3)
    d = (ri0 + ri1) / 2.0 - (rj0 + rj1) / 2.0
    return vi[:, None] * vj[:, None] * jnp.exp(-(d[:, None] ** 2) / (2.0 * sig[None, :] ** 2))


def _gatv2(x_src, x_dst, ei, Wl, bl, Wr, br, att, bias):
    hl = x_src @ Wl + bl
    hr = x_dst @ Wr + br
    s = ei[0]; t = ei[1]
    e = jax.nn.leaky_relu(hl[s] + hr[t], 0.2) @ att
    n_dst = x_dst.shape[0]
    m = jax.lax.stop_gradient(jax.ops.segment_max(e, t, num_segments=n_dst))
    m = jnp.where(jnp.isfinite(m), m, 0.0)
    ee = jnp.exp(e - m[t])
    den = jax.ops.segment_sum(ee, t, num_segments=n_dst)
    alpha = ee / (den[t] + 1e-16)
    out = jax.ops.segment_sum(alpha[:, None] * hl[s], t, num_segments=n_dst)
    return out + bias


def _sum_kernel(x_ref, o_ref):
    @pl.when(pl.program_id(0) == 0)
    def _():
        o_ref[...] = jnp.zeros_like(o_ref)
    o_ref[...] += jnp.sum(x_ref[...], axis=0, keepdims=True)


def _pallas_colsum(x):
    n, d = x.shape
    blk = 10000
    assert n % blk == 0
    grid = n // blk
    return pl.pallas_call(
        _sum_kernel,
        grid=(grid,),
        in_specs=[pl.BlockSpec((blk, d), lambda i: (i, 0))],
        out_specs=pl.BlockSpec((1, d), lambda i: (0, 0)),
        out_shape=jax.ShapeDtypeStruct((1, d), x.dtype),
    )(x)


def kernel(dopant_concs, radii, dopant_embed, interaction_embed, intraaction_embed, film_d_W0, film_d_b0, film_d_W1, film_d_b1, film_d_W2, film_d_b2, film_c_W0, film_c_b0, film_c_W1, film_c_b1, film_c_W2, film_c_b2, film_i_W0, film_i_b0, film_i_W1, film_i_b1, film_i_W2, film_i_b2, film_a_W0, film_a_b0, film_a_W1, film_a_b1, film_a_W2, film_a_b2, bn_d_g, bn_d_b, bn_i_g, bn_i_b, bn_a_g, bn_a_b, conv_Wl, conv_bl, conv_Wr, conv_br, conv_att, conv_bias, dopant_types, dopant_constraint_indices, interaction_types, interaction_type_indices, interaction_dopant_indices, intraaction_types, intraaction_type_indices, intraaction_dopant_indices, constraint_radii_idx, edge_index_d2i, edge_index_i2d, edge_index_d2a, edge_index_a2d):
    NMP = 3
    _radii = radii[constraint_radii_idx]
    dop = dopant_embed[dopant_types]
    dop = _film(dop, dopant_concs[:, None], film_d_W0, film_d_b0, film_d_W1, film_d_b1, film_d_W2, film_d_b2)
    r_nodes = _radii[dopant_constraint_indices]
    dop = _film(dop, r_nodes, film_c_W0, film_c_b0, film_c_W1, film_c_b1, film_c_W2, film_c_b2)
    dop = _bn(dop, bn_d_g, bn_d_b)
    inter = interaction_embed[interaction_types]
    inr = r_nodes[interaction_dopant_indices].reshape(-1, 4)
    ii = _interaction_block(inr[:, 0], inr[:, 1], inr[:, 2], inr[:, 3])
    cc = dopant_concs[interaction_dopant_indices]
    ii = (cc[:, 0] * cc[:, 1])[:, None] * ii
    ii = _bn(ii, bn_i_g, bn_i_b)
    inter = _film(inter, ii, film_i_W0, film_i_b0, film_i_W1, film_i_b1, film_i_W2, film_i_b2)
    intra = intraaction_embed[intraaction_types]
    anr = r_nodes[intraaction_dopant_indices].reshape(-1, 4)
    aa = _interaction_block(anr[:, 0], anr[:, 1], anr[:, 2], anr[:, 3])
    cc2 = dopant_concs[intraaction_dopant_indices]
    aa = (cc2[:, 0] * cc2[:, 1])[:, None] * aa
    aa = _bn(aa, bn_a_g, bn_a_b)
    intra = _film(intra, aa, film_a_W0, film_a_b0, film_a_W1, film_a_b1, film_a_W2, film_a_b2)
    xd = dop; xi = inter; xa = intra
    for l in range(NMP):
        oi = _gatv2(xd, xi, edge_index_d2i, conv_Wl[l, 0], conv_bl[l, 0], conv_Wr[l, 0], conv_br[l, 0], conv_att[l, 0], conv_bias[l, 0])
        od1 = _gatv2(xi, xd, edge_index_i2d, conv_Wl[l, 1], conv_bl[l, 1], conv_Wr[l, 1], conv_br[l, 1], conv_att[l, 1], conv_bias[l, 1])
        oa = _gatv2(xd, xa, edge_index_d2a, conv_Wl[l, 2], conv_bl[l, 2], conv_Wr[l, 2], conv_br[l, 2], conv_att[l, 2], conv_bias[l, 2])
        od2 = _gatv2(xa, xd, edge_index_a2d, conv_Wl[l, 3], conv_bl[l, 3], conv_Wr[l, 3], conv_br[l, 3], conv_att[l, 3], conv_bias[l, 3])
        xd = jax.nn.silu(od1 + od2)
        xi = jax.nn.silu(oi)
        xa = jax.nn.silu(oa)
    return _pallas_colsum(xd)


# speedup vs baseline: 8.1060x; 8.1060x over previous
import functools

import jax
import jax.numpy as jnp
from jax import lax
from jax.experimental import pallas as pl
from jax.experimental.pallas import tpu as pltpu
from jax.experimental.pallas import tpu_sc as plsc

N = 100000
E = 800000
D = 16
DPAD = 128
NACC = 100352
CH = 64
GCH = 128
NCHUNK = 391
EPAD = 32 * NCHUNK * CH
ACC1D = NACC * D
APT = ACC1D // 16
DPT = NACC // 16


def _edge_body(hl_hbm, hr_hbm, s_hbm, t_hbm, i16_hbm, att_hbm, g_hbm,
               out_hbm, den_hbm,
               s_idx, t_idx, i16_v, rows_l, rows_r, wbuf, ee_buf, att_v, g_v,
               zbuf, acc_s, den_s, gsem):
    c_ax = lax.axis_index("c")
    s_ax = lax.axis_index("s")
    w = c_ax * 16 + s_ax
    pltpu.sync_copy(att_hbm, att_v)
    pltpu.sync_copy(g_hbm, g_v)

    zvec = jnp.zeros((16,), jnp.float32)

    def zb(i, carry):
        zbuf[pl.ds(i * 16, 16)] = zvec
        return carry

    lax.fori_loop(0, 128, zb, None)
    a0 = s_ax * APT
    d0 = s_ax * DPT

    def zc(k, carry):
        pltpu.sync_copy(zbuf, acc_s.at[pl.ds(a0 + k * 2048, 2048)])
        pltpu.sync_copy(zbuf.at[pl.ds(0, 128)], den_s.at[pl.ds(d0 + k * 128, 128)])
        return carry

    lax.fori_loop(0, 49, zc, None)
    plsc.subcore_barrier()

    iota16 = lax.iota(jnp.int32, 16)
    att_vec = att_v[...]
    g16_vec = g_v[...]

    def chunk_body(cid, carry):
        row0 = w * NCHUNK + cid
        pltpu.sync_copy(s_hbm.at[pl.ds(row0, 1)], s_idx)
        pltpu.sync_copy(t_hbm.at[pl.ds(row0, 1)], t_idx)
        pltpu.sync_copy(i16_hbm.at[pl.ds(row0 * 16, 16)], i16_v)
        dl = pltpu.async_copy(hl_hbm.at[s_idx.at[0]], rows_l, gsem)
        dr = pltpu.async_copy(hr_hbm.at[t_idx.at[0]], rows_r, gsem)
        dl.wait()
        dr.wait()
        epr = CH // 16
        for gg in range(CH // 16):
            eevec = jnp.zeros((16,), jnp.float32)
            for kk in range(16):
                r = gg * 16 + kk
                rl = rows_l[r, pl.ds(0, D)]
                rr = rows_r[r, pl.ds(0, D)]
                x = rl + rr
                lk = jnp.maximum(x, 0.2 * x)
                e_k = jnp.sum(lk * att_vec - g16_vec)
                ee_v = jnp.exp(jnp.full((16,), e_k, jnp.float32))
                wbuf[r // epr, pl.ds((r % epr) * 16, 16)] = ee_v * rl
                eevec = jnp.where(iota16 == kk, ee_v, eevec)
            ee_buf[0, pl.ds(gg * 16, 16)] = eevec
        for g in range(16):
            pltpu.sync_copy(wbuf.at[g], acc_s.at[i16_v.at[g]], add=True)
        pltpu.sync_copy(ee_buf.at[0], den_s.at[t_idx.at[0]], add=True)
        return carry

    lax.fori_loop(0, NCHUNK, chunk_body, None)
    plsc.subcore_barrier()
    pltpu.sync_copy(acc_s.at[pl.ds(a0, APT)],
                    out_hbm.at[pl.ds(c_ax * ACC1D + a0, APT)])
    pltpu.sync_copy(den_s.at[pl.ds(d0, DPT)],
                    den_hbm.at[pl.ds(c_ax * NACC + d0, DPT)])


@functools.cache
def _make_edge_kernel():
    info = plsc.get_sparse_core_info()
    nc = info.num_cores
    mesh = plsc.VectorSubcoreMesh(core_axis_name="c", subcore_axis_name="s")
    return pl.kernel(
        _edge_body,
        out_type=[
            jax.ShapeDtypeStruct((nc * ACC1D,), jnp.float32),
            jax.ShapeDtypeStruct((nc * NACC,), jnp.float32),
        ],
        mesh=mesh,
        compiler_params=pltpu.CompilerParams(needs_layout_passes=False),
        scratch_types=[
            pltpu.VMEM((1, CH), jnp.int32),
            pltpu.VMEM((1, CH), jnp.int32),
            pltpu.VMEM((16, CH), jnp.int32),
            pltpu.VMEM((CH, DPAD), jnp.float32),
            pltpu.VMEM((CH, DPAD), jnp.float32),
            pltpu.VMEM((16, CH), jnp.float32),
            pltpu.VMEM((1, CH), jnp.float32),
            pltpu.VMEM((D,), jnp.float32),
            pltpu.VMEM((D,), jnp.float32),
            pltpu.VMEM((2048,), jnp.float32),
            pltpu.VMEM_SHARED((ACC1D,), jnp.float32),
            pltpu.VMEM_SHARED((NACC,), jnp.float32),
            pltpu.SemaphoreType.DMA,
        ],
    )


def _row_gather_body(table_hbm, idx_hbm, out_hbm, idx_v, rows_v, gsem):
    c_ax = lax.axis_index("c")
    s_ax = lax.axis_index("s")
    w = c_ax * 16 + s_ax
    nch = idx_hbm.shape[0] // 32

    def chunk_body(cid, carry):
        row0 = w * nch + cid
        pltpu.sync_copy(idx_hbm.at[pl.ds(row0, 1)], idx_v)
        pltpu.async_copy(table_hbm.at[idx_v.at[0]], rows_v, gsem).wait()
        pltpu.sync_copy(rows_v, out_hbm.at[pl.ds(row0 * 128, 128)])
        return carry

    lax.fori_loop(0, nch, chunk_body, None)


@functools.cache
def _make_row_gather(mpad):
    mesh = plsc.VectorSubcoreMesh(core_axis_name="c", subcore_axis_name="s")
    return pl.kernel(
        _row_gather_body,
        out_type=jax.ShapeDtypeStruct((mpad, DPAD), jnp.float32),
        mesh=mesh,
        compiler_params=pltpu.CompilerParams(needs_layout_passes=False),
        scratch_types=[
            pltpu.VMEM((1, GCH), jnp.int32),
            pltpu.VMEM((GCH, DPAD), jnp.float32),
            pltpu.SemaphoreType.DMA,
        ],
    )


def _sc_row_gather(table, idx):
    t, c = table.shape
    tab = jnp.pad(table, ((0, 0), (0, DPAD - c)))
    m = idx.shape[0]
    mpad = -(-m // 4096) * 4096
    idx2 = jnp.concatenate([idx.astype(jnp.int32),
                            jnp.zeros((mpad - m,), jnp.int32)])
    rows = _make_row_gather(mpad)(tab, idx2.reshape(mpad // 128, 128))
    return rows[:m]


def _one_hot_gather(table, idx):
    oh = jax.nn.one_hot(idx, table.shape[0], dtype=jnp.float32)
    return oh @ table


def _gat_sc(x_src, x_dst, s2, t2, i16, Wl, bl, Wr, br, att, bias):
    hl = x_src @ Wl + bl
    hr = x_dst @ Wr + br
    colmax = (jnp.max(jnp.abs(x_src) @ jnp.abs(Wl), axis=0) + jnp.abs(bl)
              + jnp.max(jnp.abs(x_dst) @ jnp.abs(Wr), axis=0) + jnp.abs(br))
    g = jnp.sum(jnp.abs(att) * colmax)
    g16 = jnp.full((D,), g / 16.0, jnp.float32)
    hlp = jnp.pad(hl, ((0, NACC - N), (0, DPAD - D)))
    hrp = jnp.pad(hr, ((0, NACC - N), (0, DPAD - D)))
    parts, dens = _make_edge_kernel()(hlp, hrp, s2, t2, i16, att, g16)
    num = jnp.sum(parts.reshape(-1, NACC, D), axis=0)[:N]
    den = jnp.sum(dens.reshape(-1, NACC), axis=0)[:N]
    return num / (den + 1e-30)[:, None] + bias


def _pad_edges(ei):
    s = jnp.concatenate([ei[0].astype(jnp.int32), jnp.zeros((EPAD - E,), jnp.int32)])
    t = jnp.concatenate([ei[1].astype(jnp.int32),
                         jnp.full((EPAD - E,), N, jnp.int32)])
    i16 = (t[:, None] * 16 + jnp.arange(16, dtype=jnp.int32)[None, :])
    return (s.reshape(EPAD // CH, CH), t.reshape(EPAD // CH, CH),
            i16.reshape(EPAD * 16 // CH, CH))


def _film(x, cond, W0, b0, W1, b1, W2, b2):
    h = jax.nn.relu(cond @ W0 + b0)
    h = jax.nn.relu(h @ W1 + b1)
    gb = h @ W2 + b2
    g = gb[:, :x.shape[1]]
    b = gb[:, x.shape[1]:]
    return g * x + b


def _bn(x, g, b):
    m = jnp.mean(x, axis=0)
    v = jnp.var(x, axis=0)
    return (x - m) / jnp.sqrt(v + 1e-5) * g + b


def _interaction_block(ri0, ri1, rj0, rj1, nsigma=5):
    sig = jnp.logspace(-1.0, 1.0, nsigma)
    vi = 4.0 / 3.0 * jnp.pi * (ri1 ** 3 - ri0 ** 3)
    vj = 4.0 / 3.0 * jnp.pi * (rj1 ** 3 - rj0 ** 3)
    d = (ri0 + ri1) / 2.0 - (rj0 + rj1) / 2.0
    return vi[:, None] * vj[:, None] * jnp.exp(-(d[:, None] ** 2) / (2.0 * sig[None, :] ** 2))


def _sum_kernel(x_ref, o_ref):
    @pl.when(pl.program_id(0) == 0)
    def _():
        o_ref[...] = jnp.zeros_like(o_ref)
    o_ref[...] += jnp.sum(x_ref[...], axis=0, keepdims=True)


def _pallas_colsum(x):
    n, d = x.shape
    blk = 10000
    assert n % blk == 0
    return pl.pallas_call(
        _sum_kernel,
        grid=(n // blk,),
        in_specs=[pl.BlockSpec((blk, d), lambda i: (i, 0))],
        out_specs=pl.BlockSpec((1, d), lambda i: (0, 0)),
        out_shape=jax.ShapeDtypeStruct((1, d), x.dtype),
    )(x)


def kernel(dopant_concs, radii, dopant_embed, interaction_embed, intraaction_embed, film_d_W0, film_d_b0, film_d_W1, film_d_b1, film_d_W2, film_d_b2, film_c_W0, film_c_b0, film_c_W1, film_c_b1, film_c_W2, film_c_b2, film_i_W0, film_i_b0, film_i_W1, film_i_b1, film_i_W2, film_i_b2, film_a_W0, film_a_b0, film_a_W1, film_a_b1, film_a_W2, film_a_b2, bn_d_g, bn_d_b, bn_i_g, bn_i_b, bn_a_g, bn_a_b, conv_Wl, conv_bl, conv_Wr, conv_br, conv_att, conv_bias, dopant_types, dopant_constraint_indices, interaction_types, interaction_type_indices, interaction_dopant_indices, intraaction_types, intraaction_type_indices, intraaction_dopant_indices, constraint_radii_idx, edge_index_d2i, edge_index_i2d, edge_index_d2a, edge_index_a2d):
    NMP = 3
    _radii = _one_hot_gather(radii[:, None],
                             constraint_radii_idx.reshape(-1)).reshape(-1, 2)
    dop = _one_hot_gather(dopant_embed, dopant_types)
    dop = _film(dop, dopant_concs[:, None], film_d_W0, film_d_b0, film_d_W1, film_d_b1, film_d_W2, film_d_b2)
    r_nodes = _sc_row_gather(_radii, dopant_constraint_indices)[:, :2]
    dop = _film(dop, r_nodes, film_c_W0, film_c_b0, film_c_W1, film_c_b1, film_c_W2, film_c_b2)
    dop = _bn(dop, bn_d_g, bn_d_b)
    node_tab = jnp.concatenate([dopant_concs[:, None], r_nodes], axis=1)
    inter = _one_hot_gather(interaction_embed, interaction_types)
    irows = _sc_row_gather(node_tab, interaction_dopant_indices.reshape(-1))
    cc = irows[:, 0].reshape(-1, 2)
    inr = irows[:, 1:3].reshape(-1, 4)
    ii = _interaction_block(inr[:, 0], inr[:, 1], inr[:, 2], inr[:, 3])
    ii = (cc[:, 0] * cc[:, 1])[:, None] * ii
    ii = _bn(ii, bn_i_g, bn_i_b)
    inter = _film(inter, ii, film_i_W0, film_i_b0, film_i_W1, film_i_b1, film_i_W2, film_i_b2)
    intra = _one_hot_gather(intraaction_embed, intraaction_types)
    arows = _sc_row_gather(node_tab, intraaction_dopant_indices.reshape(-1))
    cc2 = arows[:, 0].reshape(-1, 2)
    anr = arows[:, 1:3].reshape(-1, 4)
    aa = _interaction_block(anr[:, 0], anr[:, 1], anr[:, 2], anr[:, 3])
    aa = (cc2[:, 0] * cc2[:, 1])[:, None] * aa
    aa = _bn(aa, bn_a_g, bn_a_b)
    intra = _film(intra, aa, film_a_W0, film_a_b0, film_a_W1, film_a_b1, film_a_W2, film_a_b2)
    e_d2i = _pad_edges(edge_index_d2i)
    e_i2d = _pad_edges(edge_index_i2d)
    e_d2a = _pad_edges(edge_index_d2a)
    e_a2d = _pad_edges(edge_index_a2d)
    xd = dop; xi = inter; xa = intra
    for l in range(NMP):
        oi = _gat_sc(xd, xi, *e_d2i, conv_Wl[l, 0], conv_bl[l, 0], conv_Wr[l, 0], conv_br[l, 0], conv_att[l, 0], conv_bias[l, 0])
        od1 = _gat_sc(xi, xd, *e_i2d, conv_Wl[l, 1], conv_bl[l, 1], conv_Wr[l, 1], conv_br[l, 1], conv_att[l, 1], conv_bias[l, 1])
        oa = _gat_sc(xd, xa, *e_d2a, conv_Wl[l, 2], conv_bl[l, 2], conv_Wr[l, 2], conv_br[l, 2], conv_att[l, 2], conv_bias[l, 2])
        od2 = _gat_sc(xa, xd, *e_a2d, conv_Wl[l, 3], conv_bl[l, 3], conv_Wr[l, 3], conv_br[l, 3], conv_att[l, 3], conv_bias[l, 3])
        xd = jax.nn.silu(od1 + od2)
        xi = jax.nn.silu(oi)
        xa = jax.nn.silu(oa)
    return _pallas_colsum(xd)
